# baseline (speedup 1.0000x reference)
import functools

import numpy as np

import jax
import jax.numpy as jnp
from jax import lax
from jax.experimental import pallas as pl
from jax.experimental.pallas import tpu as pltpu
from jax.experimental.pallas import tpu_sc as plsc

D = 768
DW = D // 2
N_ROWS = 4 * 8192
NC, NS, L = 2, 16, 16
NW = NC * NS
ROWS_PER_W = N_ROWS // NW
C = 32
N_CHUNKS = ROWS_PER_W // C
GROUPS_PER_ROW = D // (2 * L)

_SRC_COLS = np.arange(D).reshape(GROUPS_PER_ROW, 2 * L)
_SRC_COLS = (_SRC_COLS // 32) * 32 + (_SRC_COLS % 32) // 2 + 16 * (_SRC_COLS % 2)
_SRC_COLS = _SRC_COLS.reshape(D).astype(np.int32)


def _sc_body(x_hbm, idx_hbm, table_hbm, alpha_hbm, out_hbm,
             idx_v, rows_v, x_v, alpha_v, sem_g, sem_x, sem_o):
    wid = lax.axis_index("s") * NC + lax.axis_index("c")
    w_base = wid * ROWS_PER_W

    pltpu.sync_copy(idx_hbm.at[pl.ds(w_base, ROWS_PER_W)], idx_v)
    pltpu.sync_copy(alpha_hbm, alpha_v)
    alpha = alpha_v[...]
    himask = jnp.full((L,), 0xFFFF0000, dtype=jnp.uint32)
    shift16 = jnp.full((L,), 16, dtype=jnp.uint32)

    def issue_loads(c, b):
        pltpu.make_async_copy(
            table_hbm.at[idx_v.at[pl.ds(c * C, C)]], rows_v.at[b],
            sem_g.at[b]).start()
        pltpu.make_async_copy(
            x_hbm.at[pl.ds(w_base + c * C, C), :], x_v.at[b],
            sem_x.at[b]).start()

    issue_loads(0, 0)

    @pl.loop(0, N_CHUNKS)
    def _chunk(cur):
        b = lax.rem(cur, 2)
        nb = 1 - b

        @pl.when(cur + 1 < N_CHUNKS)
        def _():
            @pl.when(cur >= 1)
            def _():
                pltpu.make_async_copy(
                    x_v.at[nb], out_hbm.at[pl.ds(0, C), :],
                    sem_o.at[nb]).wait()
            issue_loads(cur + 1, nb)

        pltpu.make_async_copy(
            table_hbm.at[idx_v.at[pl.ds(cur * C, C)]], rows_v.at[b],
            sem_g.at[b]).wait()
        pltpu.make_async_copy(
            x_hbm.at[pl.ds(0, C), :], x_v.at[b], sem_x.at[b]).wait()

        @pl.loop(0, C)
        def _row(r):
            for g in range(GROUPS_PER_ROW):
                packed = rows_v[b, r, pl.ds(g * L, L)]
                lo = lax.bitcast_convert_type(lax.shift_left(packed, shift16), jnp.float32)
                hi = lax.bitcast_convert_type(packed & himask, jnp.float32)
                sl0 = pl.ds(g * 2 * L, L)
                sl1 = pl.ds(g * 2 * L + L, L)
                x_v[b, r, sl0] = lo * alpha + x_v[b, r, sl0]
                x_v[b, r, sl1] = hi * alpha + x_v[b, r, sl1]

        pltpu.make_async_copy(
            x_v.at[b], out_hbm.at[pl.ds(w_base + cur * C, C), :],
            sem_o.at[b]).start()

    for b in (0, 1):
        pltpu.make_async_copy(
            x_v.at[b], out_hbm.at[pl.ds(0, C), :], sem_o.at[b]).wait()


@jax.jit
def _sc_call(x2, idx, table_swz, alpha16):
    mesh = plsc.VectorSubcoreMesh(
        core_axis_name="c", subcore_axis_name="s", num_cores=NC,
        num_subcores=NS)
    return pl.kernel(
        _sc_body,
        out_type=jax.ShapeDtypeStruct((N_ROWS, D), jnp.float32),
        mesh=mesh,
        scratch_types=[
            pltpu.VMEM((ROWS_PER_W,), jnp.int32),
            pltpu.VMEM((2, C, DW), jnp.uint32),
            pltpu.VMEM((2, C, D), jnp.float32),
            pltpu.VMEM((L,), jnp.float32),
            pltpu.SemaphoreType.DMA((2,)),
            pltpu.SemaphoreType.DMA((2,)),
            pltpu.SemaphoreType.DMA((2,)),
        ],
    )(x2, idx, table_swz, alpha16)


def kernel(x, pos, table, alpha):
    b, s, d = x.shape
    x2 = x.reshape(b * s, d)
    idx = pos.reshape(b * s)
    table_bf = table[:, _SRC_COLS].astype(jnp.bfloat16)
    table_swz = lax.bitcast_convert_type(
        table_bf.reshape(table.shape[0], DW, 2), jnp.uint32)
    alpha16 = jnp.broadcast_to(alpha, (L,))
    out = _sc_call(x2, idx, table_swz, alpha16)
    return out.reshape(b, s, d)


# speedup vs baseline: 2.8861x; 2.8861x over previous
import functools

import jax
import jax.numpy as jnp
from jax import lax
from jax.experimental import pallas as pl
from jax.experimental.pallas import tpu as pltpu
from jax.experimental.pallas import tpu_sc as plsc

D = 768
N_ROWS = 4 * 8192
NC, NS, L = 2, 16, 16
NW = NC * NS
ROWS_PER_W = N_ROWS // NW
C = 32
N_CHUNKS = ROWS_PER_W // C
LANES_PER_ROW = D // L


def _sc_body(x_hbm, idx_hbm, table_hbm, alpha_hbm, out_hbm,
             idx_v, rows_v, x_v, alpha_v, sem_g, sem_x, sem_o):
    wid = lax.axis_index("s") * NC + lax.axis_index("c")
    w_base = wid * ROWS_PER_W

    pltpu.sync_copy(idx_hbm.at[pl.ds(w_base, ROWS_PER_W)], idx_v)
    pltpu.sync_copy(alpha_hbm, alpha_v)
    alpha = alpha_v[...]

    def issue_loads(c, b):
        pltpu.make_async_copy(
            table_hbm.at[idx_v.at[pl.ds(c * C, C)]], rows_v.at[b],
            sem_g.at[b]).start()
        pltpu.make_async_copy(
            x_hbm.at[pl.ds(w_base + c * C, C), :], x_v.at[b],
            sem_x.at[b]).start()

    issue_loads(0, 0)

    @pl.loop(0, N_CHUNKS)
    def _chunk(cur):
        b = lax.rem(cur, 2)
        nb = 1 - b

        @pl.when(cur + 1 < N_CHUNKS)
        def _():
            @pl.when(cur >= 1)
            def _():
                pltpu.make_async_copy(
                    x_v.at[nb], out_hbm.at[pl.ds(0, C), :],
                    sem_o.at[nb]).wait()
            issue_loads(cur + 1, nb)

        pltpu.make_async_copy(
            table_hbm.at[idx_v.at[pl.ds(cur * C, C)]], rows_v.at[b],
            sem_g.at[b]).wait()
        pltpu.make_async_copy(
            x_hbm.at[pl.ds(0, C), :], x_v.at[b], sem_x.at[b]).wait()

        @pl.loop(0, C)
        def _row(r):
            for j in range(LANES_PER_ROW):
                sl = pl.ds(j * L, L)
                x_v[b, r, sl] = rows_v[b, r, sl] * alpha + x_v[b, r, sl]

        pltpu.make_async_copy(
            x_v.at[b], out_hbm.at[pl.ds(w_base + cur * C, C), :],
            sem_o.at[b]).start()

    for b in (0, 1):
        pltpu.make_async_copy(
            x_v.at[b], out_hbm.at[pl.ds(0, C), :], sem_o.at[b]).wait()


@jax.jit
def _sc_call(x2, idx, table, alpha16):
    mesh = plsc.VectorSubcoreMesh(
        core_axis_name="c", subcore_axis_name="s", num_cores=NC,
        num_subcores=NS)
    return pl.kernel(
        _sc_body,
        out_type=jax.ShapeDtypeStruct((N_ROWS, D), jnp.float32),
        mesh=mesh,
        scratch_types=[
            pltpu.VMEM((ROWS_PER_W,), jnp.int32),
            pltpu.VMEM((2, C, D), jnp.float32),
            pltpu.VMEM((2, C, D), jnp.float32),
            pltpu.VMEM((L,), jnp.float32),
            pltpu.SemaphoreType.DMA((2,)),
            pltpu.SemaphoreType.DMA((2,)),
            pltpu.SemaphoreType.DMA((2,)),
        ],
    )(x2, idx, table, alpha16)


def kernel(x, pos, table, alpha):
    b, s, d = x.shape
    x2 = x.reshape(b * s, d)
    idx = pos.reshape(b * s)
    alpha16 = jnp.broadcast_to(alpha, (L,))
    out = _sc_call(x2, idx, table, alpha16)
    return out.reshape(b, s, d)


# speedup vs baseline: 2.8970x; 1.0038x over previous
import jax
import jax.numpy as jnp
from jax import lax
from jax.experimental import pallas as pl
from jax.experimental.pallas import tpu as pltpu
from jax.experimental.pallas import tpu_sc as plsc

D = 768
N_ROWS = 4 * 8192
NC, NS, L = 2, 16, 16
NW = NC * NS
ROWS_PER_W = N_ROWS // NW
C = 32
N_CHUNKS = ROWS_PER_W // C
LANES_PER_ROW = D // L


def _sc_body(x_hbm, idx_hbm, table_hbm, alpha_hbm, out_hbm,
             idx_v, rows_v, x_v, alpha_v, sem_g, sem_x, sem_o):
    wid = lax.axis_index("s") * NC + lax.axis_index("c")
    w_base = wid * ROWS_PER_W

    pltpu.sync_copy(idx_hbm.at[pl.ds(w_base, ROWS_PER_W)], idx_v)
    pltpu.sync_copy(alpha_hbm, alpha_v)
    alpha = alpha_v[...]

    def issue_loads(c, b):
        pltpu.make_async_copy(
            table_hbm.at[idx_v.at[pl.ds(c * C, C)]], rows_v.at[b],
            sem_g.at[b]).start()
        pltpu.make_async_copy(
            x_hbm.at[pl.ds(w_base + c * C, C), :], x_v.at[b],
            sem_x.at[b]).start()

    issue_loads(0, 0)

    @pl.loop(0, N_CHUNKS)
    def _chunk(cur):
        b = lax.rem(cur, 2)
        nb = 1 - b

        @pl.when(cur + 1 < N_CHUNKS)
        def _():
            @pl.when(cur >= 1)
            def _():
                pltpu.make_async_copy(
                    x_v.at[nb], out_hbm.at[pl.ds(0, C), :],
                    sem_o.at[nb]).wait()
            issue_loads(cur + 1, nb)

        pltpu.make_async_copy(
            table_hbm.at[idx_v.at[pl.ds(cur * C, C)]], rows_v.at[b],
            sem_g.at[b]).wait()
        pltpu.make_async_copy(
            x_hbm.at[pl.ds(0, C), :], x_v.at[b], sem_x.at[b]).wait()

        @pl.loop(0, C)
        def _row(r):
            for j in range(LANES_PER_ROW):
                sl = pl.ds(j * L, L)
                x_v[b, r, sl] = rows_v[b, r, sl] * alpha + x_v[b, r, sl]

        pltpu.make_async_copy(
            x_v.at[b], out_hbm.at[pl.ds(w_base + cur * C, C), :],
            sem_o.at[b]).start()

    for b in (0, 1):
        pltpu.make_async_copy(
            x_v.at[b], out_hbm.at[pl.ds(0, C), :], sem_o.at[b]).wait()


@jax.jit
def _sc_call(x2, idx, table, alpha16):
    mesh = plsc.VectorSubcoreMesh(
        core_axis_name="c", subcore_axis_name="s", num_cores=NC,
        num_subcores=NS)
    return pl.kernel(
        _sc_body,
        out_type=jax.ShapeDtypeStruct((N_ROWS, D), jnp.float32),
        mesh=mesh,
        scratch_types=[
            pltpu.VMEM((ROWS_PER_W,), jnp.int32),
            pltpu.VMEM((2, C, D), jnp.float32),
            pltpu.VMEM((2, C, D), jnp.float32),
            pltpu.VMEM((L,), jnp.float32),
            pltpu.SemaphoreType.DMA((2,)),
            pltpu.SemaphoreType.DMA((2,)),
            pltpu.SemaphoreType.DMA((2,)),
        ],
    )(x2, idx, table, alpha16)


def kernel(x, pos, table, alpha):
    b, s, d = x.shape
    x2 = x.reshape(b * s, d)
    idx = pos.reshape(b * s)
    alpha16 = jnp.broadcast_to(alpha, (L,))
    out = _sc_call(x2, idx, table, alpha16)
    return out.reshape(b, s, d)
